# baseline (speedup 1.0000x reference)
import jax
import jax.numpy as jnp
from jax import lax
from jax.experimental import pallas as pl
from jax.experimental.pallas import tpu as pltpu
from jax.experimental.pallas import tpu_sc as plsc

B = 16384
F = 26
D = 128

NC = 2
NS = 16
NW = NC * NS
ROWS_PER_W = B // NW
LANES = 16
CHUNKS = ROWS_PER_W // LANES


def _sc_body(x_hbm, table_hbm, out_hbm, idx_v, vals_v, out_v, sem):
    wid = lax.axis_index("s") * NC + lax.axis_index("c")
    rbase = wid * ROWS_PER_W

    pltpu.sync_copy(x_hbm.at[pl.ds(rbase * F, ROWS_PER_W * F)], idx_v)

    pltpu.async_copy(table_hbm.at[idx_v], vals_v, sem).wait()

    lane_off = jnp.arange(LANES, dtype=jnp.int32) * F

    def chunk_body(v, carry):
        base = v * (LANES * F)
        acc = plsc.load_gather(vals_v, [lane_off + base])
        for f in range(1, F):
            acc = acc + plsc.load_gather(vals_v, [lane_off + (base + f)])
        out_v[pl.ds(v * LANES, LANES)] = acc
        return carry

    lax.fori_loop(0, CHUNKS, chunk_body, 0)
    pltpu.sync_copy(out_v, out_hbm.at[pl.ds(rbase, ROWS_PER_W)])


@jax.jit
def _sc_embed_sum(x_flat, table_flat):
    mesh = plsc.VectorSubcoreMesh(core_axis_name="c", subcore_axis_name="s")
    kern = pl.kernel(
        _sc_body,
        mesh=mesh,
        out_type=jax.ShapeDtypeStruct((B,), jnp.float32),
        scratch_types=[
            pltpu.VMEM((F * ROWS_PER_W,), jnp.int32),
            pltpu.VMEM((F * ROWS_PER_W,), jnp.float32),
            pltpu.VMEM((ROWS_PER_W,), jnp.float32),
            pltpu.SemaphoreType.DMA,
        ],
        compiler_params=pltpu.CompilerParams(needs_layout_passes=False),
    )
    return kern(x_flat, table_flat)


MV_BLK = 2048


def _mv_body(xc_ref, w_ref, b_ref, emb_ref, o_ref):
    o_ref[...] = (
        jnp.dot(xc_ref[...], w_ref[...], preferred_element_type=jnp.float32)
        + b_ref[0, 0]
        + emb_ref[...]
    )


@jax.jit
def _tc_finish(x_cont, w2d, b2d, emb2d):
    return pl.pallas_call(
        _mv_body,
        grid=(B // MV_BLK,),
        in_specs=[
            pl.BlockSpec((MV_BLK, D), lambda i: (i, 0)),
            pl.BlockSpec((D, 1), lambda i: (0, 0)),
            pl.BlockSpec(memory_space=pltpu.SMEM),
            pl.BlockSpec((MV_BLK, 1), lambda i: (i, 0)),
        ],
        out_specs=pl.BlockSpec((MV_BLK, 1), lambda i: (i, 0)),
        out_shape=jax.ShapeDtypeStruct((B, 1), jnp.float32),
    )(x_cont, w2d, b2d, emb2d)


def kernel(x, x_cont, table, bias, w):
    emb = _sc_embed_sum(x.astype(jnp.int32).reshape(-1), table.reshape(-1))
    return _tc_finish(x_cont, w.reshape(D, 1), bias.reshape(1, 1),
                      emb.reshape(B, 1))


# speedup vs baseline: 1.1063x; 1.1063x over previous
import jax
import jax.numpy as jnp
from jax import lax
from jax.experimental import pallas as pl
from jax.experimental.pallas import tpu as pltpu
from jax.experimental.pallas import tpu_sc as plsc

B = 16384
F = 26
D = 128

NC = 2
NS = 16
NW = NC * NS
ROWS_PER_W = B // NW
LANES = 16
CHUNKS = ROWS_PER_W // LANES


def _sc_body(xt_hbm, table_hbm, out_hbm, idx_v, vals_v, out_v, sem, sem2):
    wid = lax.axis_index("s") * NC + lax.axis_index("c")
    rbase = wid * ROWS_PER_W

    stage = [
        pltpu.async_copy(
            xt_hbm.at[pl.ds(f * B + rbase, ROWS_PER_W)],
            idx_v.at[pl.ds(f * ROWS_PER_W, ROWS_PER_W)],
            sem2,
        )
        for f in range(F)
    ]
    for c in stage:
        c.wait()

    pltpu.async_copy(table_hbm.at[idx_v], vals_v, sem).wait()

    def chunk_body(v, carry):
        off = v * LANES
        acc = vals_v[pl.ds(off, LANES)]
        for f in range(1, F):
            acc = acc + vals_v[pl.ds(f * ROWS_PER_W + off, LANES)]
        out_v[pl.ds(off, LANES)] = acc
        return carry

    lax.fori_loop(0, CHUNKS, chunk_body, 0)
    pltpu.sync_copy(out_v, out_hbm.at[pl.ds(rbase, ROWS_PER_W)])


@jax.jit
def _sc_embed_sum(xt_flat, table_flat):
    mesh = plsc.VectorSubcoreMesh(core_axis_name="c", subcore_axis_name="s")
    kern = pl.kernel(
        _sc_body,
        mesh=mesh,
        out_type=jax.ShapeDtypeStruct((B,), jnp.float32),
        scratch_types=[
            pltpu.VMEM((F * ROWS_PER_W,), jnp.int32),
            pltpu.VMEM((F * ROWS_PER_W,), jnp.float32),
            pltpu.VMEM((ROWS_PER_W,), jnp.float32),
            pltpu.SemaphoreType.DMA,
            pltpu.SemaphoreType.DMA,
        ],
    )
    return kern(xt_flat, table_flat)


MV_BLK = 2048


def _mv_body(xc_ref, w_ref, b_ref, emb_ref, o_ref):
    o_ref[...] = (
        jnp.dot(xc_ref[...], w_ref[...], preferred_element_type=jnp.float32)
        + b_ref[0, 0]
        + emb_ref[...]
    )


@jax.jit
def _tc_finish(x_cont, w2d, b2d, emb2d):
    return pl.pallas_call(
        _mv_body,
        grid=(B // MV_BLK,),
        in_specs=[
            pl.BlockSpec((MV_BLK, D), lambda i: (i, 0)),
            pl.BlockSpec((D, 1), lambda i: (0, 0)),
            pl.BlockSpec(memory_space=pltpu.SMEM),
            pl.BlockSpec((MV_BLK, 1), lambda i: (i, 0)),
        ],
        out_specs=pl.BlockSpec((MV_BLK, 1), lambda i: (i, 0)),
        out_shape=jax.ShapeDtypeStruct((B, 1), jnp.float32),
    )(x_cont, w2d, b2d, emb2d)


def kernel(x, x_cont, table, bias, w):
    xt = x.T.astype(jnp.int32).reshape(-1)
    tflat = table.T.reshape(-1)
    emb = _sc_embed_sum(xt, tflat)
    return _tc_finish(x_cont, w.reshape(D, 1), bias.reshape(1, 1),
                      emb.reshape(B, 1))


# speedup vs baseline: 1.3343x; 1.2061x over previous
import jax
import jax.numpy as jnp
from jax import lax
from jax.experimental import pallas as pl
from jax.experimental.pallas import tpu as pltpu
from jax.experimental.pallas import tpu_sc as plsc

B = 16384
F = 26
D = 128
V = 999986

NC = 2
NS = 16
NW = NC * NS
ROWS_PER_W = B // NW
IDX_PER_W = ROWS_PER_W * F
LANES = 16
CHUNKS = ROWS_PER_W // LANES


def _sc_body(xt_hbm, table_hbm, out_hbm, idx_v, vals_v, out_v, sem, sem2):
    wid = lax.axis_index("s") * NC + lax.axis_index("c")
    rbase = wid * ROWS_PER_W

    stage = [
        pltpu.async_copy(
            xt_hbm.at[pl.ds(f * B + rbase, ROWS_PER_W)],
            idx_v.at[pl.ds(f * ROWS_PER_W, ROWS_PER_W)],
            sem2,
        )
        for f in range(F)
    ]
    for c in stage:
        c.wait()

    pltpu.async_copy(table_hbm.at[idx_v], vals_v, sem).wait()

    def chunk_body(v, carry):
        off = v * LANES
        acc = vals_v[pl.ds(off, LANES)]
        for f in range(1, F):
            acc = acc + vals_v[pl.ds(f * ROWS_PER_W + off, LANES)]
        out_v[pl.ds(off, LANES)] = acc
        return carry

    lax.fori_loop(0, CHUNKS, chunk_body, 0)
    pltpu.sync_copy(out_v, out_hbm.at[pl.ds(rbase, ROWS_PER_W)])


@jax.jit
def _sc_embed_sum(xt_flat, table_flat):
    mesh = plsc.VectorSubcoreMesh(core_axis_name="c", subcore_axis_name="s")
    kern = pl.kernel(
        _sc_body,
        mesh=mesh,
        out_type=jax.ShapeDtypeStruct((B,), jnp.float32),
        scratch_types=[
            pltpu.VMEM((IDX_PER_W,), jnp.int32),
            pltpu.VMEM((IDX_PER_W,), jnp.float32),
            pltpu.VMEM((ROWS_PER_W,), jnp.float32),
            pltpu.SemaphoreType.DMA,
            pltpu.SemaphoreType.DMA,
        ],
    )
    return kern(xt_flat, table_flat)


MV_BLK = 2048


def _mv_body(xc_ref, w_ref, b_ref, o_ref):
    o_ref[...] = jnp.sum(xc_ref[...] * w_ref[...], axis=1) + b_ref[0, 0]


@jax.jit
def _tc_matvec(x_cont, w2d, b2d):
    return pl.pallas_call(
        _mv_body,
        grid=(B // MV_BLK,),
        in_specs=[
            pl.BlockSpec((MV_BLK, D), lambda i: (i, 0)),
            pl.BlockSpec((1, D), lambda i: (0, 0)),
            pl.BlockSpec(memory_space=pltpu.SMEM),
        ],
        out_specs=pl.BlockSpec((MV_BLK,), lambda i: (i,)),
        out_shape=jax.ShapeDtypeStruct((B,), jnp.float32),
    )(x_cont, w2d, b2d)


def _add_body(a_ref, b_ref, o_ref):
    o_ref[...] = a_ref[...] + b_ref[...]


@jax.jit
def _tc_add(a, b):
    return pl.pallas_call(
        _add_body,
        grid=(B // MV_BLK,),
        in_specs=[
            pl.BlockSpec((MV_BLK,), lambda i: (i,)),
            pl.BlockSpec((MV_BLK,), lambda i: (i,)),
        ],
        out_specs=pl.BlockSpec((MV_BLK,), lambda i: (i,)),
        out_shape=jax.ShapeDtypeStruct((B,), jnp.float32),
    )(a, b)


def kernel(x, x_cont, table, bias, w):
    xt = x.T.astype(jnp.int32).reshape(-1)
    mv = _tc_matvec(x_cont, w.reshape(1, D), bias.reshape(1, 1))
    emb = _sc_embed_sum(xt, table.reshape(-1))
    return _tc_add(emb, mv).reshape(B, 1)


# speedup vs baseline: 1.3898x; 1.0416x over previous
import jax
import jax.numpy as jnp
from jax import lax
from jax.experimental import pallas as pl
from jax.experimental.pallas import tpu as pltpu
from jax.experimental.pallas import tpu_sc as plsc

B = 16384
F = 26
D = 128
V = 999986

NC = 2
NS = 16
NW = NC * NS
ROWS_PER_W = B // NW
IDX_PER_W = ROWS_PER_W * F
LANES = 16
CHUNKS = ROWS_PER_W // LANES


HF = F // 2


def _sc_body(xt_hbm, table_hbm, out_hbm, idx_v, vals_v, out_v, semA, semB,
             sem2):
    wid = lax.axis_index("s") * NC + lax.axis_index("c")
    rbase = wid * ROWS_PER_W

    def stage(f):
        return pltpu.async_copy(
            xt_hbm.at[pl.ds(f * B + rbase, ROWS_PER_W)],
            idx_v.at[pl.ds(f * ROWS_PER_W, ROWS_PER_W)],
            sem2,
        )

    s1 = [stage(f) for f in range(HF)]
    for c in s1:
        c.wait()
    g1 = pltpu.async_copy(
        table_hbm.at[idx_v.at[pl.ds(0, HF * ROWS_PER_W)]],
        vals_v.at[pl.ds(0, HF * ROWS_PER_W)], semA)

    s2 = [stage(f) for f in range(HF, F)]
    for c in s2:
        c.wait()
    g2 = pltpu.async_copy(
        table_hbm.at[idx_v.at[pl.ds(HF * ROWS_PER_W, HF * ROWS_PER_W)]],
        vals_v.at[pl.ds(HF * ROWS_PER_W, HF * ROWS_PER_W)], semB)

    g1.wait()

    def chunk_body1(v, carry):
        off = v * LANES
        acc = vals_v[pl.ds(off, LANES)]
        for f in range(1, HF):
            acc = acc + vals_v[pl.ds(f * ROWS_PER_W + off, LANES)]
        out_v[pl.ds(off, LANES)] = acc
        return carry

    lax.fori_loop(0, CHUNKS, chunk_body1, 0)
    g2.wait()

    def chunk_body2(v, carry):
        off = v * LANES
        acc = out_v[pl.ds(off, LANES)]
        for f in range(HF, F):
            acc = acc + vals_v[pl.ds(f * ROWS_PER_W + off, LANES)]
        out_v[pl.ds(off, LANES)] = acc
        return carry

    lax.fori_loop(0, CHUNKS, chunk_body2, 0)
    pltpu.sync_copy(out_v, out_hbm.at[pl.ds(rbase, ROWS_PER_W)])


@jax.jit
def _sc_embed_sum(xt_flat, table_flat):
    mesh = plsc.VectorSubcoreMesh(core_axis_name="c", subcore_axis_name="s")
    kern = pl.kernel(
        _sc_body,
        mesh=mesh,
        out_type=jax.ShapeDtypeStruct((B,), jnp.float32),
        scratch_types=[
            pltpu.VMEM((IDX_PER_W,), jnp.int32),
            pltpu.VMEM((IDX_PER_W,), jnp.float32),
            pltpu.VMEM((ROWS_PER_W,), jnp.float32),
            pltpu.SemaphoreType.DMA,
            pltpu.SemaphoreType.DMA,
            pltpu.SemaphoreType.DMA,
        ],
    )
    return kern(xt_flat, table_flat)


MV_BLK = 2048


def _mv_body(xc_ref, w_ref, b_ref, o_ref):
    o_ref[...] = jnp.sum(xc_ref[...] * w_ref[...], axis=1) + b_ref[0, 0]


@jax.jit
def _tc_matvec(x_cont, w2d, b2d):
    return pl.pallas_call(
        _mv_body,
        grid=(B // MV_BLK,),
        in_specs=[
            pl.BlockSpec((MV_BLK, D), lambda i: (i, 0)),
            pl.BlockSpec((1, D), lambda i: (0, 0)),
            pl.BlockSpec(memory_space=pltpu.SMEM),
        ],
        out_specs=pl.BlockSpec((MV_BLK,), lambda i: (i,)),
        out_shape=jax.ShapeDtypeStruct((B,), jnp.float32),
    )(x_cont, w2d, b2d)


def _add_body(a_ref, b_ref, o_ref):
    o_ref[...] = a_ref[...] + b_ref[...]


@jax.jit
def _tc_add(a, b):
    return pl.pallas_call(
        _add_body,
        out_shape=jax.ShapeDtypeStruct((B,), jnp.float32),
    )(a, b)


def kernel(x, x_cont, table, bias, w):
    xt = x.T.astype(jnp.int32).reshape(-1)
    mv = _tc_matvec(x_cont, w.reshape(1, D), bias.reshape(1, 1))
    emb = _sc_embed_sum(xt, table.reshape(-1))
    return _tc_add(emb, mv).reshape(B, 1)
